# baseline (speedup 1.0000x reference)
import functools

import jax
import jax.numpy as jnp
from jax import lax
from jax.experimental import pallas as pl
from jax.experimental.pallas import tpu as pltpu
from jax.experimental.pallas import tpu_sc as plsc

N = 10000
E = 320000
D = 128
H = 256
LD = 64
NUM_SEGMENTS = 256.0

NC = 2
NS = 16
LANES = 16

LOG2PI = 1.8378770664093453


def _sc_mesh():
    return plsc.VectorSubcoreMesh(
        core_axis_name="c", subcore_axis_name="s", num_cores=NC, num_subcores=NS
    )


_EPC = E // NC
_BLK = 8
_RSTRIDE = 624
_RWIN = 640


def _tile_range(c, s, bpc):
    nhi = bpc % NS
    nlo = bpc // NS
    nb = jnp.where(s < nhi, nlo + 1, nlo)
    start = c * bpc + jnp.where(s < nhi, s * (nlo + 1), nhi * (nlo + 1) + (s - nhi) * nlo)
    return start, nb


def _load_idx(src2_hbm, dst2_hbm, swin, dwin, b0, nb, nlo):
    r0 = b0 * _BLK
    pltpu.sync_copy(src2_hbm.at[pl.ds(r0, nlo * _BLK)], swin.at[pl.ds(0, nlo * _BLK)])
    pltpu.sync_copy(dst2_hbm.at[pl.ds(r0, nlo * _BLK)], dwin.at[pl.ds(0, nlo * _BLK)])

    @pl.when(nb > nlo)
    def _():
        pltpu.sync_copy(src2_hbm.at[pl.ds(r0 + nlo * _BLK, _BLK)],
                        swin.at[pl.ds(nlo * _BLK, _BLK)])
        pltpu.sync_copy(dst2_hbm.at[pl.ds(r0 + nlo * _BLK, _BLK)],
                        dwin.at[pl.ds(nlo * _BLK, _BLK)])


_CHA = 40
_ERA = E // _CHA
_BPCA = _ERA // NC // _BLK
_NLOA = _BPCA // NS
_MAXCHA = (_NLOA + 1) * _BLK


def _agg_body(src2_hbm, dst2_hbm, x_hbm, zeros_hbm, out_hbm,
              swin, dwin, rows0, rows1, acc, g0, g1, s0, s1):
    c = lax.axis_index("c")
    s = lax.axis_index("s")
    pltpu.sync_copy(
        zeros_hbm.at[pl.ds(s * _RSTRIDE, _RWIN)], acc.at[pl.ds(s * _RSTRIDE, _RWIN)]
    )
    b0, nb = _tile_range(c, s, _BPCA)
    nchunk = nb * _BLK
    _load_idx(src2_hbm, dst2_hbm, swin, dwin, b0, nb, _NLOA)
    plsc.subcore_barrier()

    pltpu.async_copy(x_hbm.at[swin.at[0]], rows0, g0)

    def chunk(k, carry):
        @pl.when(k % 2 == 0)
        def _():
            @pl.when(k >= 1)
            def _():
                pltpu.make_async_copy(rows1, acc.at[dwin.at[k - 1]], s1).wait()

            @pl.when(k + 1 < nchunk)
            def _():
                pltpu.async_copy(x_hbm.at[swin.at[k + 1]], rows1, g1)

            pltpu.make_async_copy(x_hbm.at[swin.at[k]], rows0, g0).wait()
            pltpu.async_copy(rows0, acc.at[dwin.at[k]], s0, add=True)

        @pl.when(k % 2 == 1)
        def _():
            pltpu.make_async_copy(rows0, acc.at[dwin.at[k - 1]], s0).wait()

            @pl.when(k + 1 < nchunk)
            def _():
                pltpu.async_copy(x_hbm.at[swin.at[k + 1]], rows0, g0)

            pltpu.make_async_copy(x_hbm.at[swin.at[k]], rows1, g1).wait()
            pltpu.async_copy(rows1, acc.at[dwin.at[k]], s1, add=True)

        return carry

    lax.fori_loop(0, nchunk, chunk, 0)
    pltpu.make_async_copy(rows1, acc.at[dwin.at[nchunk - 1]], s1).wait()
    plsc.subcore_barrier()
    pltpu.sync_copy(
        acc.at[pl.ds(s * _RSTRIDE, _RWIN)], out_hbm.at[c, pl.ds(s * _RSTRIDE, _RWIN)]
    )


@functools.lru_cache(maxsize=None)
def _agg_call():
    return functools.partial(
        pl.kernel,
        out_type=jax.ShapeDtypeStruct((NC, N, D), jnp.float32),
        mesh=_sc_mesh(),
        compiler_params=pltpu.CompilerParams(
            needs_layout_passes=False, use_tc_tiling_on_sc=False
        ),
        scratch_types=[
            pltpu.VMEM((_MAXCHA, _CHA), jnp.int32),
            pltpu.VMEM((_MAXCHA, _CHA), jnp.int32),
            pltpu.VMEM((_CHA, D), jnp.float32),
            pltpu.VMEM((_CHA, D), jnp.float32),
            pltpu.VMEM_SHARED((N, D), jnp.float32),
            pltpu.SemaphoreType.DMA,
            pltpu.SemaphoreType.DMA,
            pltpu.SemaphoreType.DMA,
            pltpu.SemaphoreType.DMA,
        ],
    )(_agg_body)


_RB = 2000
_NB = N // _RB


def _dense_body(p0, p1, x, eps, w1, w2, wmu, wlv, wd, z_out, kl_out, nlp_out):
    i = pl.program_id(0)
    agg = p0[...] + p1[...]
    h = jnp.maximum(
        jnp.dot(agg, w1[...], preferred_element_type=jnp.float32)
        + jnp.dot(x[...], w2[...], preferred_element_type=jnp.float32),
        0.0,
    )
    mu = jnp.dot(h, wmu[...], preferred_element_type=jnp.float32)
    lv = jnp.clip(jnp.dot(h, wlv[...], preferred_element_type=jnp.float32), -8.0, 8.0)
    s2 = jnp.exp(lv)
    z = mu + jnp.exp(0.5 * lv) * eps[...]
    z_out[...] = z
    klb = 0.5 * jnp.sum(mu * mu + s2 - 1.0 - lv)
    xr = jnp.dot(z, wd[...], preferred_element_type=jnp.float32)
    nlb = -0.5 * jnp.sum((x[...] - xr) ** 2) - 0.5 * _RB * D * LOG2PI

    @pl.when(i == 0)
    def _():
        kl_out[0, 0] = klb
        nlp_out[0, 0] = nlb

    @pl.when(i != 0)
    def _():
        kl_out[0, 0] += klb
        nlp_out[0, 0] += nlb


def _dense_call(p0, p1, x, eps, w1, w2, wmu, wlv, wd):
    full = lambda shape: pl.BlockSpec(shape, lambda i: (0, 0))
    blk = lambda shape: pl.BlockSpec(shape, lambda i: (i, 0))
    scalar = pl.BlockSpec((1, 1), lambda i: (0, 0), memory_space=pltpu.SMEM)
    return pl.pallas_call(
        _dense_body,
        grid=(_NB,),
        in_specs=[
            blk((_RB, D)), blk((_RB, D)), blk((_RB, D)), blk((_RB, LD)),
            full((D, H)), full((D, H)), full((H, LD)), full((H, LD)), full((LD, D)),
        ],
        out_specs=[blk((_RB, LD)), scalar, scalar],
        out_shape=[
            jax.ShapeDtypeStruct((N, LD), jnp.float32),
            jax.ShapeDtypeStruct((1, 1), jnp.float32),
            jax.ShapeDtypeStruct((1, 1), jnp.float32),
        ],
    )(p0, p1, x, eps, w1, w2, wmu, wlv, wd)


_CHC = 80
_ERC = E // _CHC
_BPCC = _ERC // NC // _BLK
_NLOC = _BPCC // NS
_MAXCHC = (_NLOC + 1) * _BLK


def _edge_dot_chunk(zs, zd, pbuf, lbuf, k):
    for e in range(_CHC):
        p0 = zs[e, pl.ds(0, 16)] * zd[e, pl.ds(0, 16)]
        p1 = zs[e, pl.ds(16, 16)] * zd[e, pl.ds(16, 16)]
        p2 = zs[e, pl.ds(32, 16)] * zd[e, pl.ds(32, 16)]
        p3 = zs[e, pl.ds(48, 16)] * zd[e, pl.ds(48, 16)]
        pbuf[e, pl.ds(0, 16)] = (p0 + p1) + (p2 + p3)
    for g in range(_CHC // LANES):
        rowi = g * LANES + lax.iota(jnp.int32, LANES)
        acc = plsc.load_gather(pbuf, [rowi, jnp.zeros((LANES,), jnp.int32)])
        for j in range(1, LANES):
            acc = acc + plsc.load_gather(pbuf, [rowi, jnp.full((LANES,), j, jnp.int32)])
        lbuf[pl.ds(k * _CHC + g * LANES, LANES)] = acc


_ZRPS = N // NS


def _edge_body(src2_hbm, dst2_hbm, z_hbm, logit_hbm,
               swin, dwin, zs0, zd0, zs1, zd1, pbuf, lbuf, zsh, g0, g1):
    c = lax.axis_index("c")
    s = lax.axis_index("s")
    b0, nb = _tile_range(c, s, _BPCC)
    nchunk = nb * _BLK
    _load_idx(src2_hbm, dst2_hbm, swin, dwin, b0, nb, _NLOC)
    pltpu.sync_copy(z_hbm.at[pl.ds(s * _ZRPS, _ZRPS)], zsh.at[pl.ds(s * _ZRPS, _ZRPS)])
    plsc.subcore_barrier()

    pltpu.async_copy(zsh.at[swin.at[0]], zs0, g0)
    pltpu.async_copy(zsh.at[dwin.at[0]], zd0, g0)

    def chunk(k, carry):
        @pl.when(k % 2 == 0)
        def _():
            @pl.when(k + 1 < nchunk)
            def _():
                pltpu.async_copy(zsh.at[swin.at[k + 1]], zs1, g1)
                pltpu.async_copy(zsh.at[dwin.at[k + 1]], zd1, g1)

            pltpu.make_async_copy(z_hbm.at[swin.at[k]], zs0, g0).wait()
            pltpu.make_async_copy(z_hbm.at[dwin.at[k]], zd0, g0).wait()
            _edge_dot_chunk(zs0, zd0, pbuf, lbuf, k)

        @pl.when(k % 2 == 1)
        def _():
            @pl.when(k + 1 < nchunk)
            def _():
                pltpu.async_copy(zsh.at[swin.at[k + 1]], zs0, g0)
                pltpu.async_copy(zsh.at[dwin.at[k + 1]], zd0, g0)

            pltpu.make_async_copy(z_hbm.at[swin.at[k]], zs1, g1).wait()
            pltpu.make_async_copy(z_hbm.at[dwin.at[k]], zd1, g1).wait()
            _edge_dot_chunk(zs1, zd1, pbuf, lbuf, k)

        return carry

    lax.fori_loop(0, nchunk, chunk, 0)

    e0 = b0 * _BLK * _CHC
    nfix = _NLOC * _BLK * _CHC
    pltpu.sync_copy(lbuf.at[pl.ds(0, nfix)], logit_hbm.at[pl.ds(e0, nfix)])

    @pl.when(nchunk * _CHC > nfix)
    def _():
        pltpu.sync_copy(
            lbuf.at[pl.ds(nfix, _BLK * _CHC)],
            logit_hbm.at[pl.ds(e0 + nfix, _BLK * _CHC)],
        )


@functools.lru_cache(maxsize=None)
def _edge_call():
    return functools.partial(
        pl.kernel,
        out_type=jax.ShapeDtypeStruct((E,), jnp.float32),
        mesh=_sc_mesh(),
        compiler_params=pltpu.CompilerParams(
            needs_layout_passes=False, use_tc_tiling_on_sc=False
        ),
        scratch_types=[
            pltpu.VMEM((_MAXCHC, _CHC), jnp.int32),
            pltpu.VMEM((_MAXCHC, _CHC), jnp.int32),
            pltpu.VMEM((_CHC, LD), jnp.float32),
            pltpu.VMEM((_CHC, LD), jnp.float32),
            pltpu.VMEM((_CHC, LD), jnp.float32),
            pltpu.VMEM((_CHC, LD), jnp.float32),
            pltpu.VMEM((_CHC, 17), jnp.float32),
            pltpu.VMEM((_MAXCHC * _CHC,), jnp.float32),
            pltpu.VMEM_SHARED((N, LD), jnp.float32),
            pltpu.SemaphoreType.DMA,
            pltpu.SemaphoreType.DMA,
        ],
    )(_edge_body)


def _tail_body(l_ref, kl_ref, nlp_ref, out_ref):
    t = l_ref[...]
    elp = jnp.sum(jnp.minimum(t, 0.0) - jnp.log1p(jnp.exp(-jnp.abs(t))))
    out_ref[0, 0] = -((nlp_ref[0, 0] + elp - kl_ref[0, 0]) / NUM_SEGMENTS)


def _tail_call(logits2d, kl_s, nlp_s):
    scalar = pl.BlockSpec(memory_space=pltpu.SMEM)
    return pl.pallas_call(
        _tail_body,
        in_specs=[pl.BlockSpec(logits2d.shape, lambda: (0, 0)), scalar, scalar],
        out_specs=scalar,
        out_shape=jax.ShapeDtypeStruct((1, 1), jnp.float32),
    )(logits2d, kl_s, nlp_s)


def kernel(x, edge_index, batch, eps, W1, W2, Wmu, Wlv, Wd):
    del batch
    src = edge_index[0]
    dst = edge_index[1]
    zeros = jnp.zeros((N, D), jnp.float32)
    parts = _agg_call()(src.reshape(_ERA, _CHA), dst.reshape(_ERA, _CHA), x, zeros)
    z, kl_s, nlp_s = _dense_call(parts[0], parts[1], x, eps, W1, W2, Wmu, Wlv, Wd)
    logits = _edge_call()(src.reshape(_ERC, _CHC), dst.reshape(_ERC, _CHC), z)
    out = _tail_call(logits.reshape(E // D, D), kl_s, nlp_s)
    return out[0, 0]
